# Initial kernel scaffold; baseline (speedup 1.0000x reference)
import jax
import jax.numpy as jnp
from jax.experimental import pallas as pl


def kernel(x, edge_index, edge_weight, W1_rel, b1_rel, W1_root, W2_rel, b2_rel, W2_root):
    raise NotImplementedError("write your pallas kernel here")


# speedup vs baseline: 4.1100x; 4.1100x over previous
import functools

import jax
import jax.numpy as jnp
from jax import lax
from jax.experimental import pallas as pl
from jax.experimental.pallas import tpu as pltpu
from jax.experimental.pallas import tpu_sc as plsc

N_NODES = 10000
N_EDGES = 320000
D = 128
LANES = 16

NC = 2
NS = 16
NW = NC * NS
CHUNK = 128
NCH = -(-N_EDGES // (NW * CHUNK))
EPW = NCH * CHUNK
EPAD = NW * EPW

N_PAD = 10240
ROWS_PER_TILE = N_PAD // NS


def _sc_segment_sum(y, idx, w):
  mesh = plsc.VectorSubcoreMesh(core_axis_name="c", subcore_axis_name="s")

  @functools.partial(
      pl.kernel,
      out_type=jax.ShapeDtypeStruct((NC, N_PAD, D), jnp.float32),
      mesh=mesh,
      scratch_types=[
          pltpu.VMEM_SHARED((N_PAD, D), jnp.float32),
          pltpu.VMEM((NCH, 2, CHUNK), jnp.int32),
          pltpu.VMEM((NCH, CHUNK), jnp.float32),
          pltpu.VMEM((CHUNK, D), jnp.float32),
          pltpu.SemaphoreType.DMA,
      ],
  )
  def k(y_hbm, idx_hbm, w_hbm, out_hbm, acc, idx_all, w_all, rows, sem):
    c = lax.axis_index("c")
    s = lax.axis_index("s")

    def zero_body(rr, carry):
      for j in range(D // LANES):
        rows[rr, pl.ds(j * LANES, LANES)] = jnp.zeros((LANES,), jnp.float32)
      return carry
    lax.fori_loop(0, CHUNK, zero_body, 0)
    base = s * ROWS_PER_TILE
    for t in range(ROWS_PER_TILE // CHUNK):
      pltpu.sync_copy(rows, acc.at[pl.ds(base + t * CHUNK, CHUNK)])
    plsc.subcore_barrier()

    pltpu.sync_copy(idx_hbm.at[c, s], idx_all)
    pltpu.sync_copy(w_hbm.at[c, s], w_all)

    def chunk_body(ch, carry):
      pltpu.async_copy(y_hbm.at[idx_all.at[ch, 0]], rows, sem).wait()

      def scale_group(g, carry2):
        w16 = w_all[ch, pl.ds(g * LANES, LANES)]
        def scale_edge(i, carry3):
          lane = jnp.broadcast_to(i, (LANES,)).astype(jnp.int32)
          wb = lax.gather(
              w16, lane[:, None],
              lax.GatherDimensionNumbers(offset_dims=(),
                                         collapsed_slice_dims=(0,),
                                         start_index_map=(0,)),
              slice_sizes=(1,),
              mode=lax.GatherScatterMode.PROMISE_IN_BOUNDS)
          e = g * LANES + i
          for j in range(D // LANES):
            sl = pl.ds(j * LANES, LANES)
            rows[e, sl] = rows[e, sl] * wb
          return carry3
        lax.fori_loop(0, LANES, scale_edge, 0)
        return carry2
      lax.fori_loop(0, CHUNK // LANES, scale_group, 0)

      pltpu.sync_copy(rows, acc.at[idx_all.at[ch, 1]], add=True)
      return carry
    lax.fori_loop(0, NCH, chunk_body, 0)

    plsc.subcore_barrier()
    pltpu.sync_copy(acc.at[pl.ds(base, ROWS_PER_TILE)],
                    out_hbm.at[c, pl.ds(base, ROWS_PER_TILE)])

  return k(y, idx, w)


_BR = 400
_DOT_DN = (((1,), (1,)), ((), ()))


def _tc_prep(x, W_rel, W_root, b_rel):
  def body(x_ref, wrel_ref, wroot_ref, b_ref, y_ref, r_ref):
    xb = x_ref[...]
    y_ref[...] = lax.dot_general(xb, wrel_ref[...], _DOT_DN,
                                 preferred_element_type=jnp.float32)
    r_ref[...] = lax.dot_general(xb, wroot_ref[...], _DOT_DN,
                                 preferred_element_type=jnp.float32) + b_ref[...]

  return pl.pallas_call(
      body,
      grid=(N_NODES // _BR,),
      in_specs=[
          pl.BlockSpec((_BR, D), lambda i: (i, 0)),
          pl.BlockSpec((D, D), lambda i: (0, 0)),
          pl.BlockSpec((D, D), lambda i: (0, 0)),
          pl.BlockSpec((1, D), lambda i: (0, 0)),
      ],
      out_specs=[
          pl.BlockSpec((_BR, D), lambda i: (i, 0)),
          pl.BlockSpec((_BR, D), lambda i: (i, 0)),
      ],
      out_shape=[jax.ShapeDtypeStruct((N_NODES, D), jnp.float32)] * 2,
  )(x, W_rel, W_root, b_rel.reshape(1, D))


def _tc_combine_prep(p, r1, W_rel, W_root, b_rel):
  def body(p_ref, r1_ref, wrel_ref, wroot_ref, b_ref, y_ref, r_ref):
    h = jnp.maximum(p_ref[0] + p_ref[1] + r1_ref[...], 0.0)
    y_ref[...] = lax.dot_general(h, wrel_ref[...], _DOT_DN,
                                 preferred_element_type=jnp.float32)
    r_ref[...] = lax.dot_general(h, wroot_ref[...], _DOT_DN,
                                 preferred_element_type=jnp.float32) + b_ref[...]

  return pl.pallas_call(
      body,
      grid=(N_NODES // _BR,),
      in_specs=[
          pl.BlockSpec((NC, _BR, D), lambda i: (0, i, 0)),
          pl.BlockSpec((_BR, D), lambda i: (i, 0)),
          pl.BlockSpec((D, D), lambda i: (0, 0)),
          pl.BlockSpec((D, D), lambda i: (0, 0)),
          pl.BlockSpec((1, D), lambda i: (0, 0)),
      ],
      out_specs=[
          pl.BlockSpec((_BR, D), lambda i: (i, 0)),
          pl.BlockSpec((_BR, D), lambda i: (i, 0)),
      ],
      out_shape=[jax.ShapeDtypeStruct((N_NODES, D), jnp.float32)] * 2,
  )(p, r1, W_rel, W_root, b_rel.reshape(1, D))


def _tc_final(q, r2):
  def body(q_ref, r2_ref, out_ref):
    out_ref[...] = q_ref[0] + q_ref[1] + r2_ref[...]

  return pl.pallas_call(
      body,
      grid=(N_NODES // _BR,),
      in_specs=[
          pl.BlockSpec((NC, _BR, D), lambda i: (0, i, 0)),
          pl.BlockSpec((_BR, D), lambda i: (i, 0)),
      ],
      out_specs=pl.BlockSpec((_BR, D), lambda i: (i, 0)),
      out_shape=jax.ShapeDtypeStruct((N_NODES, D), jnp.float32),
  )(q, r2)


def kernel(x, edge_index, edge_weight, W1_rel, b1_rel, W1_root,
           W2_rel, b2_rel, W2_root):
  pad = EPAD - N_EDGES
  src = jnp.pad(edge_index[0].astype(jnp.int32), (0, pad))
  dst = jnp.pad(edge_index[1].astype(jnp.int32), (0, pad))
  w = jnp.pad(edge_weight.astype(jnp.float32), (0, pad))
  idx = jnp.stack(
      [src.reshape(NC, NS, NCH, CHUNK), dst.reshape(NC, NS, NCH, CHUNK)],
      axis=3)
  w = w.reshape(NC, NS, NCH, CHUNK)

  y1, r1 = _tc_prep(x, W1_rel, W1_root, b1_rel)
  p1 = _sc_segment_sum(y1, idx, w)[:, :N_NODES]
  y2, r2 = _tc_combine_prep(p1, r1, W2_rel, W2_root, b2_rel)
  p2 = _sc_segment_sum(y2, idx, w)[:, :N_NODES]
  return _tc_final(p2, r2)
